# baseline (speedup 1.0000x reference)
import functools

import jax
import jax.numpy as jnp
from jax import lax
from jax.experimental import pallas as pl
from jax.experimental.pallas import tpu as pltpu
from jax.experimental.pallas import tpu_sc as plsc

_NC = 2
_NS = 16


def _mm_bias_block(x_ref, w_ref, b_ref, o_ref):
    o_ref[...] = (
        jnp.dot(x_ref[...], w_ref[...], preferred_element_type=jnp.float32)
        + b_ref[...]
    )


def _merge_block(s_ref, a_ref, b_ref, o_ref):
    t = a_ref[...] + b_ref[...]
    t = jnp.where(t >= 0.0, t, 0.01 * t)
    o_ref[...] = t * s_ref[0, 0]


def _merge_mm_block(a_ref, b_ref, m_ref, c_ref, o_ref):
    t = a_ref[...] + b_ref[...]
    t = jnp.where(t >= 0.0, t, 0.01 * t)
    o_ref[...] = (
        jnp.dot(t, m_ref[...], preferred_element_type=jnp.float32)
        + c_ref[...]
    )


def _in_transform(x, w_t, b):
    n, d = x.shape
    h = w_t.shape[1]
    bn = 2000
    return pl.pallas_call(
        _mm_bias_block,
        grid=(n // bn,),
        in_specs=[
            pl.BlockSpec((bn, d), lambda i: (i, 0)),
            pl.BlockSpec((d, h), lambda i: (0, 0)),
            pl.BlockSpec((1, h), lambda i: (0, 0)),
        ],
        out_specs=pl.BlockSpec((bn, h), lambda i: (i, 0)),
        out_shape=jax.ShapeDtypeStruct((n, h), jnp.float32),
    )(x, w_t, b)


def _merge(s, p0, p1):
    n, h = p0.shape
    bn = 2000
    return pl.pallas_call(
        _merge_block,
        grid=(n // bn,),
        in_specs=[
            pl.BlockSpec(memory_space=pltpu.SMEM),
            pl.BlockSpec((bn, h), lambda i: (i, 0)),
            pl.BlockSpec((bn, h), lambda i: (i, 0)),
        ],
        out_specs=pl.BlockSpec((bn, h), lambda i: (i, 0)),
        out_shape=jax.ShapeDtypeStruct((n, h), jnp.float32),
    )(s, p0, p1)


def _final(p0, p1, m_t, c):
    n, h = p0.shape
    o = m_t.shape[1]
    bn = 2000
    return pl.pallas_call(
        _merge_mm_block,
        grid=(n // bn,),
        in_specs=[
            pl.BlockSpec((bn, h), lambda i: (i, 0)),
            pl.BlockSpec((bn, h), lambda i: (i, 0)),
            pl.BlockSpec((h, o), lambda i: (0, 0)),
            pl.BlockSpec((1, o), lambda i: (0, 0)),
        ],
        out_specs=pl.BlockSpec((bn, o), lambda i: (i, 0)),
        out_shape=jax.ShapeDtypeStruct((n, o), jnp.float32),
    )(p0, p1, m_t, c)


def _sc_propagate(table, src4, dst4, zeros_pad):
    n, h = table.shape
    nb_blk, bpb, k = src4.shape[1], src4.shape[2], src4.shape[3]
    ch = nb_blk * bpb
    n_acc = zeros_pad.shape[0]
    rpt = (-(-n // _NS) + 7) // 8 * 8
    rpt_last = n - (_NS - 1) * rpt
    assert rpt_last > 0 and rpt_last % 8 == 0

    mesh = plsc.VectorSubcoreMesh(core_axis_name="c", subcore_axis_name="s")

    @functools.partial(
        pl.kernel,
        mesh=mesh,
        out_type=[
            jax.ShapeDtypeStruct((n, h), jnp.float32),
            jax.ShapeDtypeStruct((n, h), jnp.float32),
        ],
        scratch_types=[
            pltpu.VMEM((2, bpb, k), jnp.int32),
            pltpu.VMEM((2, bpb, k), jnp.int32),
            pltpu.VMEM((2, k, h), jnp.float32),
            pltpu.VMEM_SHARED((n_acc, h), jnp.float32),
            pltpu.SemaphoreType.DMA,
            pltpu.SemaphoreType.DMA,
            pltpu.SemaphoreType.DMA,
            pltpu.SemaphoreType.DMA,
        ],
    )
    def run(table_hbm, src_hbm, dst_hbm, z_hbm, p0_hbm, p1_hbm,
            src_v, dst_v, rows_v, acc_sh, sem_g, sem_s, sem_is, sem_id):
        cid = lax.axis_index("c")
        sid = lax.axis_index("s")
        wid = cid * _NS + sid

        @pl.when(sid < _NS - 1)
        def _():
            pltpu.sync_copy(z_hbm.at[pl.ds(sid * rpt, rpt)],
                            acc_sh.at[pl.ds(sid * rpt, rpt)])

        @pl.when(sid == _NS - 1)
        def _():
            last = n_acc - (_NS - 1) * rpt
            pltpu.sync_copy(z_hbm.at[pl.ds((_NS - 1) * rpt, last)],
                            acc_sh.at[pl.ds((_NS - 1) * rpt, last)])

        pltpu.async_copy(src_hbm.at[wid, 0], src_v.at[0], sem_is)
        pltpu.async_copy(dst_hbm.at[wid, 0], dst_v.at[0], sem_id)
        plsc.subcore_barrier()
        pltpu.make_async_copy(src_hbm.at[wid, 0], src_v.at[0], sem_is).wait()
        pltpu.make_async_copy(dst_hbm.at[wid, 0], dst_v.at[0], sem_id).wait()

        assert ch % 2 == 0

        def body(j, carry):
            m = j // bpb
            c = j - m * bpb
            mb = lax.rem(m, 2)

            @pl.when(jnp.logical_and(c == 0, m >= 1))
            def _():
                pltpu.make_async_copy(src_hbm.at[wid, m],
                                      src_v.at[mb], sem_is).wait()
                pltpu.make_async_copy(dst_hbm.at[wid, m],
                                      dst_v.at[mb], sem_id).wait()

            @pl.when(jnp.logical_and(c == 0, m + 1 < nb_blk))
            def _():
                pltpu.async_copy(src_hbm.at[wid, m + 1],
                                 src_v.at[1 - mb], sem_is)
                pltpu.async_copy(dst_hbm.at[wid, m + 1],
                                 dst_v.at[1 - mb], sem_id)

            pltpu.sync_copy(table_hbm.at[src_v.at[mb, c]], rows_v.at[0])
            pltpu.sync_copy(rows_v.at[0], acc_sh.at[dst_v.at[mb, c]],
                            add=True)
            return carry

        lax.fori_loop(0, ch, body, 0)
        plsc.subcore_barrier()

        for core, out_hbm in ((0, p0_hbm), (1, p1_hbm)):
            @pl.when(jnp.logical_and(cid == core, sid < _NS - 1))
            def _(out_hbm=out_hbm):
                pltpu.sync_copy(acc_sh.at[pl.ds(sid * rpt, rpt)],
                                out_hbm.at[pl.ds(sid * rpt, rpt)])

            @pl.when(jnp.logical_and(cid == core, sid == _NS - 1))
            def _(out_hbm=out_hbm):
                pltpu.sync_copy(acc_sh.at[pl.ds((_NS - 1) * rpt, rpt_last)],
                                out_hbm.at[pl.ds((_NS - 1) * rpt, rpt_last)])

    return run(table, src4, dst4, zeros_pad)


def kernel(x, edge_index, W_in, b_in, plasticity, syn, in_proj_w, in_proj_b,
           out_proj_w, out_proj_b, W_out, b_out):
    n, d = x.shape
    h = W_in.shape[0]
    e = edge_index.shape[1]
    nw = _NC * _NS
    k = 128
    bpb = 8
    ept = -(-e // (nw * bpb * k)) * bpb * k
    nb_blk = ept // (bpb * k)
    pad = nw * ept - e

    sig = jax.nn.sigmoid
    gate = sig(plasticity) * sig(syn)

    w_in_t = (W_in * gate[0]).T
    b0 = (b_in * gate[0]).reshape(1, h)

    w_v = in_proj_w[2 * h:]
    b_v = in_proj_b[2 * h:]
    m_t = (W_out @ out_proj_w @ w_v).T
    c = ((b_v @ out_proj_w.T + out_proj_b) @ W_out.T + b_out).reshape(1, -1)

    ei = edge_index.astype(jnp.int32)
    src4 = jnp.concatenate(
        [ei[0], jnp.zeros((pad,), jnp.int32)]).reshape(nw, nb_blk, bpb, k)
    trash = n + (jnp.arange(pad, dtype=jnp.int32) % k)
    dst4 = jnp.concatenate([ei[1], trash]).reshape(nw, nb_blk, bpb, k)
    z = jnp.zeros((n + k, h), jnp.float32)

    table = _in_transform(x, w_in_t, b0)
    p0, p1 = _sc_propagate(table, src4, dst4, z)
    table = _merge(gate[1].reshape(1, 1), p0, p1)
    p0, p1 = _sc_propagate(table, src4, dst4, z)
    table = _merge(gate[2].reshape(1, 1), p0, p1)
    p0, p1 = _sc_propagate(table, src4, dst4, z)
    return _final(p0, p1, m_t, c)


# speedup vs baseline: 3.8216x; 3.8216x over previous
import functools

import jax
import jax.numpy as jnp
from jax import lax
from jax.experimental import pallas as pl
from jax.experimental.pallas import tpu as pltpu
from jax.experimental.pallas import tpu_sc as plsc

_NC = 2
_NS = 16


def _mm_bias_block(x_ref, w_ref, b_ref, o_ref):
    o_ref[...] = (
        jnp.dot(x_ref[...], w_ref[...], preferred_element_type=jnp.float32)
        + b_ref[...]
    )


def _merge_block(s_ref, a_ref, b_ref, o_ref):
    t = a_ref[...] + b_ref[...]
    t = jnp.where(t >= 0.0, t, 0.01 * t)
    o_ref[...] = t * s_ref[0, 0]


def _merge_mm_block(a_ref, b_ref, m_ref, c_ref, o_ref):
    t = a_ref[...] + b_ref[...]
    t = jnp.where(t >= 0.0, t, 0.01 * t)
    o_ref[...] = (
        jnp.dot(t, m_ref[...], preferred_element_type=jnp.float32)
        + c_ref[...]
    )


def _in_transform(x, w_t, b):
    n, d = x.shape
    h = w_t.shape[1]
    bn = 2000
    return pl.pallas_call(
        _mm_bias_block,
        grid=(n // bn,),
        in_specs=[
            pl.BlockSpec((bn, d), lambda i: (i, 0)),
            pl.BlockSpec((d, h), lambda i: (0, 0)),
            pl.BlockSpec((1, h), lambda i: (0, 0)),
        ],
        out_specs=pl.BlockSpec((bn, h), lambda i: (i, 0)),
        out_shape=jax.ShapeDtypeStruct((n, h), jnp.float32),
    )(x, w_t, b)


def _merge(s, p0, p1):
    n, h = p0.shape
    bn = 2000
    return pl.pallas_call(
        _merge_block,
        grid=(n // bn,),
        in_specs=[
            pl.BlockSpec(memory_space=pltpu.SMEM),
            pl.BlockSpec((bn, h), lambda i: (i, 0)),
            pl.BlockSpec((bn, h), lambda i: (i, 0)),
        ],
        out_specs=pl.BlockSpec((bn, h), lambda i: (i, 0)),
        out_shape=jax.ShapeDtypeStruct((n, h), jnp.float32),
    )(s, p0, p1)


def _final(p0, p1, m_t, c):
    n, h = p0.shape
    o = m_t.shape[1]
    bn = 2000
    return pl.pallas_call(
        _merge_mm_block,
        grid=(n // bn,),
        in_specs=[
            pl.BlockSpec((bn, h), lambda i: (i, 0)),
            pl.BlockSpec((bn, h), lambda i: (i, 0)),
            pl.BlockSpec((h, o), lambda i: (0, 0)),
            pl.BlockSpec((1, o), lambda i: (0, 0)),
        ],
        out_specs=pl.BlockSpec((bn, o), lambda i: (i, 0)),
        out_shape=jax.ShapeDtypeStruct((n, o), jnp.float32),
    )(p0, p1, m_t, c)


def _sc_propagate(table, src4, dst4, zeros_pad):
    n, h = table.shape
    nb_blk, bpb, k = src4.shape[1], src4.shape[2], src4.shape[3]
    ch = nb_blk * bpb
    n_acc = zeros_pad.shape[0]
    rpt = (-(-n // _NS) + 7) // 8 * 8
    rpt_last = n - (_NS - 1) * rpt
    assert rpt_last > 0 and rpt_last % 8 == 0

    mesh = plsc.VectorSubcoreMesh(core_axis_name="c", subcore_axis_name="s")

    @functools.partial(
        pl.kernel,
        mesh=mesh,
        out_type=[
            jax.ShapeDtypeStruct((n, h), jnp.float32),
            jax.ShapeDtypeStruct((n, h), jnp.float32),
        ],
        scratch_types=[
            pltpu.VMEM((2, bpb, k), jnp.int32),
            pltpu.VMEM((2, bpb, k), jnp.int32),
            pltpu.VMEM((2, k, h), jnp.float32),
            pltpu.VMEM_SHARED((n_acc, h), jnp.float32),
            pltpu.SemaphoreType.DMA,
            pltpu.SemaphoreType.DMA,
            pltpu.SemaphoreType.DMA,
            pltpu.SemaphoreType.DMA,
        ],
    )
    def run(table_hbm, src_hbm, dst_hbm, z_hbm, p0_hbm, p1_hbm,
            src_v, dst_v, rows_v, acc_sh, sem_g, sem_s, sem_is, sem_id):
        cid = lax.axis_index("c")
        sid = lax.axis_index("s")
        wid = cid * _NS + sid

        @pl.when(sid < _NS - 1)
        def _():
            pltpu.sync_copy(z_hbm.at[pl.ds(sid * rpt, rpt)],
                            acc_sh.at[pl.ds(sid * rpt, rpt)])

        @pl.when(sid == _NS - 1)
        def _():
            last = n_acc - (_NS - 1) * rpt
            pltpu.sync_copy(z_hbm.at[pl.ds((_NS - 1) * rpt, last)],
                            acc_sh.at[pl.ds((_NS - 1) * rpt, last)])

        pltpu.async_copy(src_hbm.at[wid, 0], src_v.at[0], sem_is)
        pltpu.async_copy(dst_hbm.at[wid, 0], dst_v.at[0], sem_id)
        plsc.subcore_barrier()
        pltpu.make_async_copy(src_hbm.at[wid, 0], src_v.at[0], sem_is).wait()
        pltpu.make_async_copy(dst_hbm.at[wid, 0], dst_v.at[0], sem_id).wait()

        pltpu.async_copy(table_hbm.at[src_v.at[0, 0]], rows_v.at[0], sem_g)

        def _and(a, bb):
            return jnp.logical_and(a, bb)

        def when(cond):
            if isinstance(cond, bool):
                cond = jnp.bool_(cond)
            return pl.when(cond)

        def step(j, b):
            nb = 1 - b
            m = j // bpb
            c = j - m * bpb
            mb = m % 2 if isinstance(j, int) else lax.rem(m, 2)

            @when(j >= 1)
            def _():
                jp = j - 1
                mp = jp // bpb
                mpb = mp % 2 if isinstance(j, int) else lax.rem(mp, 2)
                pltpu.make_async_copy(
                    rows_v.at[nb],
                    acc_sh.at[dst_v.at[mpb, jp - mp * bpb]],
                    sem_s).wait()

            @when(_and(c == 0, m + 1 < nb_blk))
            def _():
                pltpu.async_copy(src_hbm.at[wid, m + 1],
                                 src_v.at[1 - mb], sem_is)
                pltpu.async_copy(dst_hbm.at[wid, m + 1],
                                 dst_v.at[1 - mb], sem_id)

            @when(j + 1 < ch)
            def _():
                jn = j + 1
                mn = jn // bpb
                mnb = mn % 2 if isinstance(j, int) else lax.rem(mn, 2)

                @when(jn - mn * bpb == 0)
                def _():
                    pltpu.make_async_copy(src_hbm.at[wid, mn],
                                          src_v.at[mnb], sem_is).wait()
                    pltpu.make_async_copy(dst_hbm.at[wid, mn],
                                          dst_v.at[mnb], sem_id).wait()

                pltpu.async_copy(
                    table_hbm.at[src_v.at[mnb, jn - mn * bpb]],
                    rows_v.at[nb], sem_g)

            pltpu.make_async_copy(
                table_hbm.at[src_v.at[mb, c]], rows_v.at[b], sem_g).wait()
            pltpu.async_copy(
                rows_v.at[b], acc_sh.at[dst_v.at[mb, c]], sem_s, add=True)

        def body(jj, carry):
            step(2 * jj, 0)
            step(2 * jj + 1, 1)
            return carry

        lax.fori_loop(0, ch // 2, body, 0)
        if ch % 2 == 1:
            step(ch - 1, 0)
        mlast = (ch - 1) // bpb
        pltpu.make_async_copy(
            rows_v.at[(ch - 1) % 2],
            acc_sh.at[dst_v.at[mlast % 2, (ch - 1) - mlast * bpb]],
            sem_s).wait()
        plsc.subcore_barrier()

        for core, out_hbm in ((0, p0_hbm), (1, p1_hbm)):
            @pl.when(jnp.logical_and(cid == core, sid < _NS - 1))
            def _(out_hbm=out_hbm):
                pltpu.sync_copy(acc_sh.at[pl.ds(sid * rpt, rpt)],
                                out_hbm.at[pl.ds(sid * rpt, rpt)])

            @pl.when(jnp.logical_and(cid == core, sid == _NS - 1))
            def _(out_hbm=out_hbm):
                pltpu.sync_copy(acc_sh.at[pl.ds((_NS - 1) * rpt, rpt_last)],
                                out_hbm.at[pl.ds((_NS - 1) * rpt, rpt_last)])

    return run(table, src4, dst4, zeros_pad)


def kernel(x, edge_index, W_in, b_in, plasticity, syn, in_proj_w, in_proj_b,
           out_proj_w, out_proj_b, W_out, b_out):
    n, d = x.shape
    h = W_in.shape[0]
    e = edge_index.shape[1]
    nw = _NC * _NS
    k = 80
    bpb = 5
    ept = -(-e // (nw * bpb * k)) * bpb * k
    nb_blk = ept // (bpb * k)
    pad = nw * ept - e

    sig = jax.nn.sigmoid
    gate = sig(plasticity) * sig(syn)

    w_in_t = (W_in * gate[0]).T
    b0 = (b_in * gate[0]).reshape(1, h)

    w_v = in_proj_w[2 * h:]
    b_v = in_proj_b[2 * h:]
    m_t = (W_out @ out_proj_w @ w_v).T
    c = ((b_v @ out_proj_w.T + out_proj_b) @ W_out.T + b_out).reshape(1, -1)

    ei = edge_index.astype(jnp.int32)
    src4 = jnp.concatenate(
        [ei[0], jnp.zeros((pad,), jnp.int32)]).reshape(nw, nb_blk, bpb, k)
    trash = n + (jnp.arange(pad, dtype=jnp.int32) % k)
    dst4 = jnp.concatenate([ei[1], trash]).reshape(nw, nb_blk, bpb, k)
    extra = ((k + 7) // 8 * 8) if pad else 0
    z = jnp.zeros((n + extra, h), jnp.float32)

    table = _in_transform(x, w_in_t, b0)
    p0, p1 = _sc_propagate(table, src4, dst4, z)
    table = _merge(gate[1].reshape(1, 1), p0, p1)
    p0, p1 = _sc_propagate(table, src4, dst4, z)
    table = _merge(gate[2].reshape(1, 1), p0, p1)
    p0, p1 = _sc_propagate(table, src4, dst4, z)
    return _final(p0, p1, m_t, c)


# speedup vs baseline: 4.2487x; 1.1118x over previous
import functools

import jax
import jax.numpy as jnp
from jax import lax
from jax.experimental import pallas as pl
from jax.experimental.pallas import tpu as pltpu
from jax.experimental.pallas import tpu_sc as plsc

_NC = 2
_NS = 16
_DEPTH = 4


def _mm_bias_block(x_ref, w_ref, b_ref, o_ref):
    o_ref[...] = (
        jnp.dot(x_ref[...], w_ref[...], preferred_element_type=jnp.float32)
        + b_ref[...]
    )


def _merge_block(s_ref, a_ref, b_ref, o_ref):
    t = a_ref[...] + b_ref[...]
    t = jnp.where(t >= 0.0, t, 0.01 * t)
    o_ref[...] = t * s_ref[0, 0]


def _merge_mm_block(a_ref, b_ref, m_ref, c_ref, o_ref):
    t = a_ref[...] + b_ref[...]
    t = jnp.where(t >= 0.0, t, 0.01 * t)
    o_ref[...] = (
        jnp.dot(t, m_ref[...], preferred_element_type=jnp.float32)
        + c_ref[...]
    )


def _in_transform(x, w_t, b):
    n, d = x.shape
    h = w_t.shape[1]
    bn = 2000
    return pl.pallas_call(
        _mm_bias_block,
        grid=(n // bn,),
        in_specs=[
            pl.BlockSpec((bn, d), lambda i: (i, 0)),
            pl.BlockSpec((d, h), lambda i: (0, 0)),
            pl.BlockSpec((1, h), lambda i: (0, 0)),
        ],
        out_specs=pl.BlockSpec((bn, h), lambda i: (i, 0)),
        out_shape=jax.ShapeDtypeStruct((n, h), jnp.float32),
    )(x, w_t, b)


def _merge(s, p0, p1):
    n, h = p0.shape
    bn = 2000
    return pl.pallas_call(
        _merge_block,
        grid=(n // bn,),
        in_specs=[
            pl.BlockSpec(memory_space=pltpu.SMEM),
            pl.BlockSpec((bn, h), lambda i: (i, 0)),
            pl.BlockSpec((bn, h), lambda i: (i, 0)),
        ],
        out_specs=pl.BlockSpec((bn, h), lambda i: (i, 0)),
        out_shape=jax.ShapeDtypeStruct((n, h), jnp.float32),
    )(s, p0, p1)


def _final(p0, p1, m_t, c):
    n, h = p0.shape
    o = m_t.shape[1]
    bn = 2000
    return pl.pallas_call(
        _merge_mm_block,
        grid=(n // bn,),
        in_specs=[
            pl.BlockSpec((bn, h), lambda i: (i, 0)),
            pl.BlockSpec((bn, h), lambda i: (i, 0)),
            pl.BlockSpec((h, o), lambda i: (0, 0)),
            pl.BlockSpec((1, o), lambda i: (0, 0)),
        ],
        out_specs=pl.BlockSpec((bn, o), lambda i: (i, 0)),
        out_shape=jax.ShapeDtypeStruct((n, o), jnp.float32),
    )(p0, p1, m_t, c)


def _sc_propagate(table, src4, dst4, zeros_pad):
    n, h = table.shape
    nb_blk, bpb, k = src4.shape[1], src4.shape[2], src4.shape[3]
    ch = nb_blk * bpb
    n_acc = zeros_pad.shape[0]
    rpt = (-(-n // _NS) + 7) // 8 * 8
    rpt_last = n - (_NS - 1) * rpt
    assert rpt_last > 0 and rpt_last % 8 == 0

    mesh = plsc.VectorSubcoreMesh(core_axis_name="c", subcore_axis_name="s")

    @functools.partial(
        pl.kernel,
        mesh=mesh,
        out_type=[
            jax.ShapeDtypeStruct((n, h), jnp.float32),
            jax.ShapeDtypeStruct((n, h), jnp.float32),
        ],
        scratch_types=[
            pltpu.VMEM((2, bpb, k), jnp.int32),
            pltpu.VMEM((2, bpb, k), jnp.int32),
            pltpu.VMEM((_DEPTH, k, h), jnp.float32),
            pltpu.VMEM_SHARED((n_acc, h), jnp.float32),
            pltpu.SemaphoreType.DMA,
            pltpu.SemaphoreType.DMA,
            pltpu.SemaphoreType.DMA,
            pltpu.SemaphoreType.DMA,
        ],
    )
    def run(table_hbm, src_hbm, dst_hbm, z_hbm, p0_hbm, p1_hbm,
            src_v, dst_v, rows_v, acc_sh, sem_g, sem_s, sem_is, sem_id):
        cid = lax.axis_index("c")
        sid = lax.axis_index("s")
        wid = cid * _NS + sid

        @pl.when(sid < _NS - 1)
        def _():
            pltpu.sync_copy(z_hbm.at[pl.ds(sid * rpt, rpt)],
                            acc_sh.at[pl.ds(sid * rpt, rpt)])

        @pl.when(sid == _NS - 1)
        def _():
            last = n_acc - (_NS - 1) * rpt
            pltpu.sync_copy(z_hbm.at[pl.ds((_NS - 1) * rpt, last)],
                            acc_sh.at[pl.ds((_NS - 1) * rpt, last)])

        pltpu.async_copy(src_hbm.at[wid, 0], src_v.at[0], sem_is)
        pltpu.async_copy(dst_hbm.at[wid, 0], dst_v.at[0], sem_id)
        plsc.subcore_barrier()
        pltpu.make_async_copy(src_hbm.at[wid, 0], src_v.at[0], sem_is).wait()
        pltpu.make_async_copy(dst_hbm.at[wid, 0], dst_v.at[0], sem_id).wait()

        pltpu.async_copy(table_hbm.at[src_v.at[0, 0]], rows_v.at[0], sem_g)

        def _and(a, bb):
            return jnp.logical_and(a, bb)

        def when(cond):
            if isinstance(cond, bool):
                cond = jnp.bool_(cond)
            return pl.when(cond)

        dep = _DEPTH

        def mc_of(j):
            m = j // bpb
            c = j - m * bpb
            mb = m % 2 if isinstance(j, int) else lax.rem(m, 2)
            return m, c, mb

        def step(j, b):
            nb = (b + 1) % dep
            m, c, mb = mc_of(j)

            @when(j + 1 >= dep)
            def _():
                jp = j + 1 - dep
                mp, cp, mpb = mc_of(jp)
                pltpu.make_async_copy(
                    rows_v.at[nb], acc_sh.at[dst_v.at[mpb, cp]],
                    sem_s).wait()

            @when(_and(c == 0, m + 1 < nb_blk))
            def _():
                pltpu.async_copy(src_hbm.at[wid, m + 1],
                                 src_v.at[1 - mb], sem_is)
                pltpu.async_copy(dst_hbm.at[wid, m + 1],
                                 dst_v.at[1 - mb], sem_id)

            @when(j + 1 < ch)
            def _():
                jn = j + 1
                mn, cn, mnb = mc_of(jn)

                @when(cn == 0)
                def _():
                    pltpu.make_async_copy(src_hbm.at[wid, mn],
                                          src_v.at[mnb], sem_is).wait()
                    pltpu.make_async_copy(dst_hbm.at[wid, mn],
                                          dst_v.at[mnb], sem_id).wait()

                pltpu.async_copy(
                    table_hbm.at[src_v.at[mnb, cn]], rows_v.at[nb], sem_g)

            pltpu.make_async_copy(
                table_hbm.at[src_v.at[mb, c]], rows_v.at[b], sem_g).wait()
            pltpu.async_copy(
                rows_v.at[b], acc_sh.at[dst_v.at[mb, c]], sem_s, add=True)

        def body(jj, carry):
            for t in range(dep):
                step(dep * jj + t, t)
            return carry

        lax.fori_loop(0, ch // dep, body, 0)
        for j in range((ch // dep) * dep, ch):
            step(j, j % dep)
        for j in range(max(0, ch - dep + 1), ch):
            m, c, mb = mc_of(j)
            pltpu.make_async_copy(
                rows_v.at[j % dep], acc_sh.at[dst_v.at[mb, c]],
                sem_s).wait()
        plsc.subcore_barrier()

        for core, out_hbm in ((0, p0_hbm), (1, p1_hbm)):
            @pl.when(jnp.logical_and(cid == core, sid < _NS - 1))
            def _(out_hbm=out_hbm):
                pltpu.sync_copy(acc_sh.at[pl.ds(sid * rpt, rpt)],
                                out_hbm.at[pl.ds(sid * rpt, rpt)])

            @pl.when(jnp.logical_and(cid == core, sid == _NS - 1))
            def _(out_hbm=out_hbm):
                pltpu.sync_copy(acc_sh.at[pl.ds((_NS - 1) * rpt, rpt_last)],
                                out_hbm.at[pl.ds((_NS - 1) * rpt, rpt_last)])

    return run(table, src4, dst4, zeros_pad)


def kernel(x, edge_index, W_in, b_in, plasticity, syn, in_proj_w, in_proj_b,
           out_proj_w, out_proj_b, W_out, b_out):
    n, d = x.shape
    h = W_in.shape[0]
    e = edge_index.shape[1]
    nw = _NC * _NS
    k = 80
    bpb = 5
    ept = -(-e // (nw * bpb * k)) * bpb * k
    nb_blk = ept // (bpb * k)
    pad = nw * ept - e

    sig = jax.nn.sigmoid
    gate = sig(plasticity) * sig(syn)

    w_in_t = (W_in * gate[0]).T
    b0 = (b_in * gate[0]).reshape(1, h)

    w_v = in_proj_w[2 * h:]
    b_v = in_proj_b[2 * h:]
    m_t = (W_out @ out_proj_w @ w_v).T
    c = ((b_v @ out_proj_w.T + out_proj_b) @ W_out.T + b_out).reshape(1, -1)

    ei = edge_index.astype(jnp.int32)
    src4 = jnp.concatenate(
        [ei[0], jnp.zeros((pad,), jnp.int32)]).reshape(nw, nb_blk, bpb, k)
    trash = n + (jnp.arange(pad, dtype=jnp.int32) % k)
    dst4 = jnp.concatenate([ei[1], trash]).reshape(nw, nb_blk, bpb, k)
    extra = ((k + 7) // 8 * 8) if pad else 0
    z = jnp.zeros((n + extra, h), jnp.float32)

    table = _in_transform(x, w_in_t, b0)
    p0, p1 = _sc_propagate(table, src4, dst4, z)
    table = _merge(gate[1].reshape(1, 1), p0, p1)
    p0, p1 = _sc_propagate(table, src4, dst4, z)
    table = _merge(gate[2].reshape(1, 1), p0, p1)
    p0, p1 = _sc_propagate(table, src4, dst4, z)
    return _final(p0, p1, m_t, c)


# speedup vs baseline: 4.6830x; 1.1022x over previous
import functools

import jax
import jax.numpy as jnp
from jax import lax
from jax.experimental import pallas as pl
from jax.experimental.pallas import tpu as pltpu
from jax.experimental.pallas import tpu_sc as plsc

_NC = 2
_NS = 16
_DEPTH = 4


def _mm_bias_block(x_ref, w_ref, b_ref, o_ref):
    o_ref[...] = (
        jnp.dot(x_ref[...], w_ref[...], preferred_element_type=jnp.float32)
        + b_ref[...]
    )


def _merge_block(s_ref, a_ref, b_ref, o_ref):
    t = a_ref[...] + b_ref[...]
    t = jnp.where(t >= 0.0, t, 0.01 * t)
    o_ref[...] = t * s_ref[0, 0]


def _merge_mm_block(a_ref, b_ref, m_ref, c_ref, o_ref):
    t = a_ref[...] + b_ref[...]
    t = jnp.where(t >= 0.0, t, 0.01 * t)
    o_ref[...] = (
        jnp.dot(t, m_ref[...], preferred_element_type=jnp.float32)
        + c_ref[...]
    )


def _in_transform(x, w_t, b):
    n, d = x.shape
    h = w_t.shape[1]
    bn = 2000
    return pl.pallas_call(
        _mm_bias_block,
        grid=(n // bn,),
        in_specs=[
            pl.BlockSpec((bn, d), lambda i: (i, 0)),
            pl.BlockSpec((d, h), lambda i: (0, 0)),
            pl.BlockSpec((1, h), lambda i: (0, 0)),
        ],
        out_specs=pl.BlockSpec((bn, h), lambda i: (i, 0)),
        out_shape=jax.ShapeDtypeStruct((n, h), jnp.float32),
    )(x, w_t, b)


def _merge(s, p0, p1):
    n, h = p0.shape
    bn = 2000
    return pl.pallas_call(
        _merge_block,
        grid=(n // bn,),
        in_specs=[
            pl.BlockSpec(memory_space=pltpu.SMEM),
            pl.BlockSpec((bn, h), lambda i: (i, 0)),
            pl.BlockSpec((bn, h), lambda i: (i, 0)),
        ],
        out_specs=pl.BlockSpec((bn, h), lambda i: (i, 0)),
        out_shape=jax.ShapeDtypeStruct((n, h), jnp.float32),
    )(s, p0, p1)


def _final(p0, p1, m_t, c):
    n, h = p0.shape
    o = m_t.shape[1]
    bn = 2000
    return pl.pallas_call(
        _merge_mm_block,
        grid=(n // bn,),
        in_specs=[
            pl.BlockSpec((bn, h), lambda i: (i, 0)),
            pl.BlockSpec((bn, h), lambda i: (i, 0)),
            pl.BlockSpec((h, o), lambda i: (0, 0)),
            pl.BlockSpec((1, o), lambda i: (0, 0)),
        ],
        out_specs=pl.BlockSpec((bn, o), lambda i: (i, 0)),
        out_shape=jax.ShapeDtypeStruct((n, o), jnp.float32),
    )(p0, p1, m_t, c)


def _sc_propagate(table, src4, dst4, zeros_pad):
    n, h = table.shape
    nb_blk, bpb, k = src4.shape[1], src4.shape[2], src4.shape[3]
    ch = nb_blk * bpb
    n_acc = zeros_pad.shape[0]
    rpt = (-(-n // _NS) + 7) // 8 * 8
    rpt_last = n - (_NS - 1) * rpt
    assert rpt_last > 0 and rpt_last % 8 == 0

    mesh = plsc.VectorSubcoreMesh(core_axis_name="c", subcore_axis_name="s")

    @functools.partial(
        pl.kernel,
        mesh=mesh,
        out_type=[
            jax.ShapeDtypeStruct((n, h), jnp.float32),
            jax.ShapeDtypeStruct((n, h), jnp.float32),
        ],
        scratch_types=[
            pltpu.VMEM((4, bpb, k), jnp.int32),
            pltpu.VMEM((4, bpb, k), jnp.int32),
            pltpu.VMEM((_DEPTH, k, h), jnp.float32),
            pltpu.VMEM_SHARED((n_acc, h), jnp.float32),
            pltpu.SemaphoreType.DMA,
            pltpu.SemaphoreType.DMA,
            pltpu.SemaphoreType.DMA,
            pltpu.SemaphoreType.DMA,
        ],
    )
    def run(table_hbm, src_hbm, dst_hbm, z_hbm, p0_hbm, p1_hbm,
            src_v, dst_v, rows_v, acc_sh, sem_g, sem_s, sem_is, sem_id):
        cid = lax.axis_index("c")
        sid = lax.axis_index("s")
        wid = cid * _NS + sid

        @pl.when(sid < _NS - 1)
        def _():
            pltpu.sync_copy(z_hbm.at[pl.ds(sid * rpt, rpt)],
                            acc_sh.at[pl.ds(sid * rpt, rpt)])

        @pl.when(sid == _NS - 1)
        def _():
            last = n_acc - (_NS - 1) * rpt
            pltpu.sync_copy(z_hbm.at[pl.ds((_NS - 1) * rpt, last)],
                            acc_sh.at[pl.ds((_NS - 1) * rpt, last)])

        pltpu.async_copy(src_hbm.at[wid, 0], src_v.at[0], sem_is)
        pltpu.async_copy(dst_hbm.at[wid, 0], dst_v.at[0], sem_id)
        plsc.subcore_barrier()
        pltpu.make_async_copy(src_hbm.at[wid, 0], src_v.at[0], sem_is).wait()
        pltpu.make_async_copy(dst_hbm.at[wid, 0], dst_v.at[0], sem_id).wait()

        def _and(a, bb):
            return jnp.logical_and(a, bb)

        def when(cond):
            if isinstance(cond, bool):
                cond = jnp.bool_(cond)
            return pl.when(cond)

        dep = _DEPTH
        assert dep == 4 and ch >= 2

        def mc_of(j):
            m = j // bpb
            c = j - m * bpb
            mb = m % 4 if isinstance(j, int) else lax.rem(m, 4)
            return m, c, mb

        pltpu.async_copy(table_hbm.at[src_v.at[0, 0]], rows_v.at[0], sem_g)
        pltpu.async_copy(table_hbm.at[src_v.at[0, 1 % bpb]],
                         rows_v.at[1], sem_g)

        def step(j, b):
            nb2 = (b + 2) % dep
            m, c, mb = mc_of(j)

            @when(j >= 2)
            def _():
                jp = j - 2
                mp, cp, mpb = mc_of(jp)
                pltpu.make_async_copy(
                    rows_v.at[nb2], acc_sh.at[dst_v.at[mpb, cp]],
                    sem_s).wait()

            @when(_and(c == 0, m + 1 < nb_blk))
            def _():
                tb = (m + 1) % 4 if isinstance(j, int) else lax.rem(m + 1, 4)
                pltpu.async_copy(src_hbm.at[wid, m + 1],
                                 src_v.at[tb], sem_is)
                pltpu.async_copy(dst_hbm.at[wid, m + 1],
                                 dst_v.at[tb], sem_id)

            @when(j + 2 < ch)
            def _():
                jn = j + 2
                mn, cn, mnb = mc_of(jn)

                @when(cn == 0)
                def _():
                    pltpu.make_async_copy(src_hbm.at[wid, mn],
                                          src_v.at[mnb], sem_is).wait()
                    pltpu.make_async_copy(dst_hbm.at[wid, mn],
                                          dst_v.at[mnb], sem_id).wait()

                pltpu.async_copy(
                    table_hbm.at[src_v.at[mnb, cn]], rows_v.at[nb2], sem_g)

            pltpu.make_async_copy(
                table_hbm.at[src_v.at[mb, c]], rows_v.at[b], sem_g).wait()
            pltpu.async_copy(
                rows_v.at[b], acc_sh.at[dst_v.at[mb, c]], sem_s, add=True)

        def body(jj, carry):
            for t in range(dep):
                step(dep * jj + t, t)
            return carry

        lax.fori_loop(0, ch // dep, body, 0)
        for j in range((ch // dep) * dep, ch):
            step(j, j % dep)
        for j in range(max(0, ch - 2), ch):
            m, c, mb = mc_of(j)
            pltpu.make_async_copy(
                rows_v.at[j % dep], acc_sh.at[dst_v.at[mb, c]],
                sem_s).wait()
        plsc.subcore_barrier()

        for core, out_hbm in ((0, p0_hbm), (1, p1_hbm)):
            @pl.when(jnp.logical_and(cid == core, sid < _NS - 1))
            def _(out_hbm=out_hbm):
                pltpu.sync_copy(acc_sh.at[pl.ds(sid * rpt, rpt)],
                                out_hbm.at[pl.ds(sid * rpt, rpt)])

            @pl.when(jnp.logical_and(cid == core, sid == _NS - 1))
            def _(out_hbm=out_hbm):
                pltpu.sync_copy(acc_sh.at[pl.ds((_NS - 1) * rpt, rpt_last)],
                                out_hbm.at[pl.ds((_NS - 1) * rpt, rpt_last)])

    return run(table, src4, dst4, zeros_pad)


def kernel(x, edge_index, W_in, b_in, plasticity, syn, in_proj_w, in_proj_b,
           out_proj_w, out_proj_b, W_out, b_out):
    n, d = x.shape
    h = W_in.shape[0]
    e = edge_index.shape[1]
    nw = _NC * _NS
    k = 80
    bpb = 5
    ept = -(-e // (nw * bpb * k)) * bpb * k
    nb_blk = ept // (bpb * k)
    pad = nw * ept - e

    sig = jax.nn.sigmoid
    gate = sig(plasticity) * sig(syn)

    w_in_t = (W_in * gate[0]).T
    b0 = (b_in * gate[0]).reshape(1, h)

    w_v = in_proj_w[2 * h:]
    b_v = in_proj_b[2 * h:]
    m_t = (W_out @ out_proj_w @ w_v).T
    c = ((b_v @ out_proj_w.T + out_proj_b) @ W_out.T + b_out).reshape(1, -1)

    ei = edge_index.astype(jnp.int32)
    src4 = jnp.concatenate(
        [ei[0], jnp.zeros((pad,), jnp.int32)]).reshape(nw, nb_blk, bpb, k)
    trash = n + (jnp.arange(pad, dtype=jnp.int32) % k)
    dst4 = jnp.concatenate([ei[1], trash]).reshape(nw, nb_blk, bpb, k)
    extra = ((k + 7) // 8 * 8) if pad else 0
    z = jnp.zeros((n + extra, h), jnp.float32)

    table = _in_transform(x, w_in_t, b0)
    p0, p1 = _sc_propagate(table, src4, dst4, z)
    table = _merge(gate[1].reshape(1, 1), p0, p1)
    p0, p1 = _sc_propagate(table, src4, dst4, z)
    table = _merge(gate[2].reshape(1, 1), p0, p1)
    p0, p1 = _sc_propagate(table, src4, dst4, z)
    return _final(p0, p1, m_t, c)


# speedup vs baseline: 4.7512x; 1.0146x over previous
import functools

import jax
import jax.numpy as jnp
from jax import lax
from jax.experimental import pallas as pl
from jax.experimental.pallas import tpu as pltpu
from jax.experimental.pallas import tpu_sc as plsc

_NC = 2
_NS = 16
_DEPTH = 4


def _mm_bias_block(x_ref, w_ref, b_ref, o_ref):
    o_ref[...] = (
        jnp.dot(x_ref[...], w_ref[...], preferred_element_type=jnp.float32)
        + b_ref[...]
    )


def _merge_block(s_ref, a_ref, b_ref, o_ref):
    t = a_ref[...] + b_ref[...]
    t = jnp.where(t >= 0.0, t, 0.01 * t)
    o_ref[...] = t * s_ref[0, 0]


def _merge_mm_block(a_ref, b_ref, m_ref, c_ref, o_ref):
    t = a_ref[...] + b_ref[...]
    t = jnp.where(t >= 0.0, t, 0.01 * t)
    o_ref[...] = (
        jnp.dot(t, m_ref[...], preferred_element_type=jnp.float32)
        + c_ref[...]
    )


def _in_transform(x, w_t, b):
    n, d = x.shape
    h = w_t.shape[1]
    bn = 2000
    return pl.pallas_call(
        _mm_bias_block,
        grid=(n // bn,),
        in_specs=[
            pl.BlockSpec((bn, d), lambda i: (i, 0)),
            pl.BlockSpec((d, h), lambda i: (0, 0)),
            pl.BlockSpec((1, h), lambda i: (0, 0)),
        ],
        out_specs=pl.BlockSpec((bn, h), lambda i: (i, 0)),
        out_shape=jax.ShapeDtypeStruct((n, h), jnp.float32),
    )(x, w_t, b)


def _merge(s, p0, p1):
    n, h = p0.shape
    bn = 2000
    return pl.pallas_call(
        _merge_block,
        grid=(n // bn,),
        in_specs=[
            pl.BlockSpec(memory_space=pltpu.SMEM),
            pl.BlockSpec((bn, h), lambda i: (i, 0)),
            pl.BlockSpec((bn, h), lambda i: (i, 0)),
        ],
        out_specs=pl.BlockSpec((bn, h), lambda i: (i, 0)),
        out_shape=jax.ShapeDtypeStruct((n, h), jnp.float32),
    )(s, p0, p1)


def _final(p0, p1, m_t, c):
    n, h = p0.shape
    o = m_t.shape[1]
    bn = 2000
    return pl.pallas_call(
        _merge_mm_block,
        grid=(n // bn,),
        in_specs=[
            pl.BlockSpec((bn, h), lambda i: (i, 0)),
            pl.BlockSpec((bn, h), lambda i: (i, 0)),
            pl.BlockSpec((h, o), lambda i: (0, 0)),
            pl.BlockSpec((1, o), lambda i: (0, 0)),
        ],
        out_specs=pl.BlockSpec((bn, o), lambda i: (i, 0)),
        out_shape=jax.ShapeDtypeStruct((n, o), jnp.float32),
    )(p0, p1, m_t, c)


def _sc_propagate(table, src4, dst4, zeros_pad):
    n, h = table.shape
    nb_blk, bpb, k = src4.shape[1], src4.shape[2], src4.shape[3]
    ch = nb_blk * bpb
    n_acc = zeros_pad.shape[0]
    rpt = (-(-n // _NS) + 7) // 8 * 8
    rpt_last = n - (_NS - 1) * rpt
    assert rpt_last > 0 and rpt_last % 8 == 0

    mesh = plsc.VectorSubcoreMesh(core_axis_name="c", subcore_axis_name="s")

    @functools.partial(
        pl.kernel,
        mesh=mesh,
        out_type=[
            jax.ShapeDtypeStruct((n, h), jnp.float32),
            jax.ShapeDtypeStruct((n, h), jnp.float32),
        ],
        scratch_types=[
            pltpu.VMEM((4, bpb, k), jnp.int32),
            pltpu.VMEM((4, bpb, k), jnp.int32),
            pltpu.VMEM((_DEPTH, k, h), jnp.float32),
            pltpu.VMEM_SHARED((n_acc, h), jnp.float32),
            pltpu.SemaphoreType.DMA,
            pltpu.SemaphoreType.DMA,
            pltpu.SemaphoreType.DMA,
            pltpu.SemaphoreType.DMA,
            pltpu.SemaphoreType.DMA,
        ],
    )
    def run(table_hbm, src_hbm, dst_hbm, z_hbm, p0_hbm, p1_hbm,
            src_v, dst_v, rows_v, acc_sh, sem_g, sem_s, sem_is, sem_id,
            sem_z):
        cid = lax.axis_index("c")
        sid = lax.axis_index("s")
        wid = cid * _NS + sid

        pltpu.async_copy(src_hbm.at[wid, 0], src_v.at[0], sem_is)
        pltpu.async_copy(dst_hbm.at[wid, 0], dst_v.at[0], sem_id)
        last = n_acc - (_NS - 1) * rpt

        @pl.when(sid < _NS - 1)
        def _():
            pltpu.async_copy(z_hbm.at[pl.ds(sid * rpt, rpt)],
                             acc_sh.at[pl.ds(sid * rpt, rpt)], sem_z)

        @pl.when(sid == _NS - 1)
        def _():
            pltpu.async_copy(z_hbm.at[pl.ds((_NS - 1) * rpt, last)],
                             acc_sh.at[pl.ds((_NS - 1) * rpt, last)], sem_z)

        pltpu.make_async_copy(src_hbm.at[wid, 0], src_v.at[0], sem_is).wait()
        pltpu.make_async_copy(dst_hbm.at[wid, 0], dst_v.at[0], sem_id).wait()

        def _and(a, bb):
            return jnp.logical_and(a, bb)

        def when(cond):
            if isinstance(cond, bool):
                cond = jnp.bool_(cond)
            return pl.when(cond)

        dep = _DEPTH
        assert dep == 4 and ch >= 2

        def mc_of(j):
            m = j // bpb
            c = j - m * bpb
            mb = m % 4 if isinstance(j, int) else lax.rem(m, 4)
            return m, c, mb

        pltpu.async_copy(table_hbm.at[src_v.at[0, 0]], rows_v.at[0], sem_g)
        pltpu.async_copy(table_hbm.at[src_v.at[0, 1 % bpb]],
                         rows_v.at[1], sem_g)

        @pl.when(sid < _NS - 1)
        def _():
            pltpu.make_async_copy(z_hbm.at[pl.ds(sid * rpt, rpt)],
                                  acc_sh.at[pl.ds(sid * rpt, rpt)],
                                  sem_z).wait()

        @pl.when(sid == _NS - 1)
        def _():
            pltpu.make_async_copy(z_hbm.at[pl.ds((_NS - 1) * rpt, last)],
                                  acc_sh.at[pl.ds((_NS - 1) * rpt, last)],
                                  sem_z).wait()
        plsc.subcore_barrier()

        def step(j, b):
            nb2 = (b + 2) % dep
            m, c, mb = mc_of(j)

            @when(j >= 2)
            def _():
                jp = j - 2
                mp, cp, mpb = mc_of(jp)
                pltpu.make_async_copy(
                    rows_v.at[nb2], acc_sh.at[dst_v.at[mpb, cp]],
                    sem_s).wait()

            @when(_and(c == 0, m + 1 < nb_blk))
            def _():
                tb = (m + 1) % 4 if isinstance(j, int) else lax.rem(m + 1, 4)
                pltpu.async_copy(src_hbm.at[wid, m + 1],
                                 src_v.at[tb], sem_is)
                pltpu.async_copy(dst_hbm.at[wid, m + 1],
                                 dst_v.at[tb], sem_id)

            @when(j + 2 < ch)
            def _():
                jn = j + 2
                mn, cn, mnb = mc_of(jn)

                @when(cn == 0)
                def _():
                    pltpu.make_async_copy(src_hbm.at[wid, mn],
                                          src_v.at[mnb], sem_is).wait()
                    pltpu.make_async_copy(dst_hbm.at[wid, mn],
                                          dst_v.at[mnb], sem_id).wait()

                pltpu.async_copy(
                    table_hbm.at[src_v.at[mnb, cn]], rows_v.at[nb2], sem_g)

            pltpu.make_async_copy(
                table_hbm.at[src_v.at[mb, c]], rows_v.at[b], sem_g).wait()
            pltpu.async_copy(
                rows_v.at[b], acc_sh.at[dst_v.at[mb, c]], sem_s, add=True)

        def body(jj, carry):
            for t in range(dep):
                step(dep * jj + t, t)
            return carry

        lax.fori_loop(0, ch // dep, body, 0)
        for j in range((ch // dep) * dep, ch):
            step(j, j % dep)
        for j in range(max(0, ch - 2), ch):
            m, c, mb = mc_of(j)
            pltpu.make_async_copy(
                rows_v.at[j % dep], acc_sh.at[dst_v.at[mb, c]],
                sem_s).wait()
        plsc.subcore_barrier()

        for core, out_hbm in ((0, p0_hbm), (1, p1_hbm)):
            @pl.when(jnp.logical_and(cid == core, sid < _NS - 1))
            def _(out_hbm=out_hbm):
                pltpu.sync_copy(acc_sh.at[pl.ds(sid * rpt, rpt)],
                                out_hbm.at[pl.ds(sid * rpt, rpt)])

            @pl.when(jnp.logical_and(cid == core, sid == _NS - 1))
            def _(out_hbm=out_hbm):
                pltpu.sync_copy(acc_sh.at[pl.ds((_NS - 1) * rpt, rpt_last)],
                                out_hbm.at[pl.ds((_NS - 1) * rpt, rpt_last)])

    return run(table, src4, dst4, zeros_pad)


def kernel(x, edge_index, W_in, b_in, plasticity, syn, in_proj_w, in_proj_b,
           out_proj_w, out_proj_b, W_out, b_out):
    n, d = x.shape
    h = W_in.shape[0]
    e = edge_index.shape[1]
    nw = _NC * _NS
    k = 80
    bpb = 5
    ept = -(-e // (nw * bpb * k)) * bpb * k
    nb_blk = ept // (bpb * k)
    pad = nw * ept - e

    sig = jax.nn.sigmoid
    gate = sig(plasticity) * sig(syn)

    w_in_t = (W_in * gate[0]).T
    b0 = (b_in * gate[0]).reshape(1, h)

    w_v = in_proj_w[2 * h:]
    b_v = in_proj_b[2 * h:]
    m_t = (W_out @ out_proj_w @ w_v).T
    c = ((b_v @ out_proj_w.T + out_proj_b) @ W_out.T + b_out).reshape(1, -1)

    ei = edge_index.astype(jnp.int32)
    src4 = jnp.concatenate(
        [ei[0], jnp.zeros((pad,), jnp.int32)]).reshape(nw, nb_blk, bpb, k)
    trash = n + (jnp.arange(pad, dtype=jnp.int32) % k)
    dst4 = jnp.concatenate([ei[1], trash]).reshape(nw, nb_blk, bpb, k)
    extra = ((k + 7) // 8 * 8) if pad else 0
    z = jnp.zeros((n + extra, h), jnp.float32)

    table = _in_transform(x, w_in_t, b0)
    p0, p1 = _sc_propagate(table, src4, dst4, z)
    table = _merge(gate[1].reshape(1, 1), p0, p1)
    p0, p1 = _sc_propagate(table, src4, dst4, z)
    table = _merge(gate[2].reshape(1, 1), p0, p1)
    p0, p1 = _sc_propagate(table, src4, dst4, z)
    return _final(p0, p1, m_t, c)
